# Initial kernel scaffold; baseline (speedup 1.0000x reference)
import jax
import jax.numpy as jnp
from jax.experimental import pallas as pl


def kernel(x, edge_index, Wl1, bl1, Wr1, Wl2, bl2, Wr2):
    raise NotImplementedError("write your pallas kernel here")


# speedup vs baseline: 3.3480x; 3.3480x over previous
import functools

import jax
import jax.numpy as jnp
from jax import lax
from jax.experimental import pallas as pl
from jax.experimental.pallas import tpu as pltpu
from jax.experimental.pallas import tpu_sc as plsc

N = 10000
D = 128
NC = 2
NS = 16
NW = NC * NS
K = 128
G = 8
CW = 16
RPAD = 10112
DUMMY = N

_mesh = plsc.VectorSubcoreMesh(core_axis_name="c", subcore_axis_name="s")


def _agg_body(ng, feat, srcw, dstw, zrows, out_acc,
              sb0, db0, sb1, db1, ra, rb, sga, sgb, sst, acc):
    c = lax.axis_index("c")
    s = lax.axis_index("s")
    wid = c * NS + s

    zn = RPAD // NS
    z0 = s * zn
    pltpu.sync_copy(zrows.at[pl.ds(z0, zn)], acc.at[pl.ds(z0, zn)])

    pltpu.sync_copy(srcw.at[wid, 0], sb0)
    pltpu.sync_copy(dstw.at[wid, 0], db0)
    plsc.subcore_barrier()
    pltpu.async_copy(feat.at[sb0.at[0]], ra, sga)

    def run_group(gp, sb, db, sb_nxt, db_nxt, second):
        for j in range(G):
            buf, sem = (ra, sga) if j % 2 == 0 else (rb, sgb)
            obuf, osem = (rb, sgb) if j % 2 == 0 else (ra, sga)
            if j < G - 1:
                pltpu.async_copy(feat.at[sb.at[j + 1]], obuf, osem)
            elif not second:
                pltpu.make_async_copy(srcw.at[wid, 0], sb_nxt, sst).wait()
                pltpu.make_async_copy(dstw.at[wid, 0], db_nxt, sst).wait()
                pltpu.async_copy(feat.at[sb_nxt.at[0]], obuf, osem)
            else:

                @pl.when(gp < ng // 2 - 1)
                def _():
                    pltpu.make_async_copy(srcw.at[wid, 0], sb_nxt, sst).wait()
                    pltpu.make_async_copy(dstw.at[wid, 0], db_nxt, sst).wait()
                    pltpu.async_copy(feat.at[sb_nxt.at[0]], obuf, osem)

            pltpu.make_async_copy(feat.at[sb.at[j]], buf, sem).wait()
            pltpu.sync_copy(buf, acc.at[db.at[j]], add=True)

    def pair(gp, carry):
        g0 = 2 * gp
        pltpu.async_copy(srcw.at[wid, g0 + 1], sb1, sst)
        pltpu.async_copy(dstw.at[wid, g0 + 1], db1, sst)
        run_group(gp, sb0, db0, sb1, db1, second=False)

        @pl.when(gp < ng // 2 - 1)
        def _():
            pltpu.async_copy(srcw.at[wid, g0 + 2], sb0, sst)
            pltpu.async_copy(dstw.at[wid, g0 + 2], db0, sst)

        run_group(gp, sb1, db1, sb0, db0, second=True)
        return carry

    lax.fori_loop(0, ng // 2, pair, 0)
    plsc.subcore_barrier()

    rn = RPAD // NS
    r0 = s * rn
    pltpu.sync_copy(acc.at[pl.ds(r0, rn)], out_acc.at[c, pl.ds(r0, rn)])


def _make_agg(ng):
    scratch = [
        pltpu.VMEM((G, K), jnp.int32),
        pltpu.VMEM((G, K), jnp.int32),
        pltpu.VMEM((G, K), jnp.int32),
        pltpu.VMEM((G, K), jnp.int32),
        pltpu.VMEM((K, D), jnp.float32),
        pltpu.VMEM((K, D), jnp.float32),
        pltpu.SemaphoreType.DMA,
        pltpu.SemaphoreType.DMA,
        pltpu.SemaphoreType.DMA,
        pltpu.VMEM_SHARED((RPAD, D), jnp.float32),
    ]
    return pl.kernel(
        functools.partial(_agg_body, ng),
        out_type=jax.ShapeDtypeStruct((NC, RPAD, D), jnp.float32),
        mesh=_mesh,
        scratch_types=scratch,
    )


def _cnt_body(ng, dstw, zrows, ones_hbm, out_cnt, dfull, ones_v, sd, cnt):
    c = lax.axis_index("c")
    s = lax.axis_index("s")
    wid = c * NS + s

    zn = RPAD // NS
    z0 = s * zn
    pltpu.sync_copy(zrows.at[pl.ds(z0, zn)], cnt.at[pl.ds(z0, zn)])
    pltpu.sync_copy(ones_hbm, ones_v)
    pltpu.sync_copy(dstw.at[wid], dfull)
    plsc.subcore_barrier()

    def group(g, carry):
        for j in range(G):
            pltpu.async_copy(ones_v, cnt.at[dfull.at[g, j]], sd, add=True)
        for j in range(G):
            pltpu.make_async_copy(ones_v, cnt.at[dfull.at[g, j]], sd).wait()
        return carry

    lax.fori_loop(0, ng, group, 0)
    plsc.subcore_barrier()

    rn = RPAD // NS
    r0 = s * rn
    pltpu.sync_copy(cnt.at[pl.ds(r0, rn)], out_cnt.at[c, pl.ds(r0, rn)])


def _make_cnt(ng):
    scratch = [
        pltpu.VMEM((ng, G, K), jnp.int32),
        pltpu.VMEM((K, D), jnp.float32),
        pltpu.SemaphoreType.DMA,
        pltpu.VMEM_SHARED((RPAD, D), jnp.float32),
    ]
    return pl.kernel(
        functools.partial(_cnt_body, ng),
        out_type=jax.ShapeDtypeStruct((NC, RPAD, D), jnp.float32),
        mesh=_mesh,
        scratch_types=scratch,
    )


def _dense_body(relu, a0, a1, c0, c1, x_ref, wl, bl, wr, o_ref):
    agg = a0[0] + a1[0]
    cnt = (c0[0] + c1[0])[:, 0:1]
    mean = agg / jnp.maximum(cnt, 1.0)
    acc = lax.dot_general(mean, wl[...], (((1,), (1,)), ((), ())),
                          precision=lax.Precision.HIGHEST)
    acc += lax.dot_general(x_ref[...], wr[...], (((1,), (1,)), ((), ())),
                           precision=lax.Precision.HIGHEST)
    acc += bl[...]
    o_ref[...] = jnp.maximum(acc, 0.0) if relu else acc


def _dense_layer(aggp, cntp, x, wl, bl, wr, relu):
    bm = 1000
    grid = (N // bm,)
    return pl.pallas_call(
        functools.partial(_dense_body, relu),
        grid=grid,
        in_specs=[
            pl.BlockSpec((1, bm, D), lambda i: (0, i, 0)),
            pl.BlockSpec((1, bm, D), lambda i: (1, i, 0)),
            pl.BlockSpec((1, bm, D), lambda i: (0, i, 0)),
            pl.BlockSpec((1, bm, D), lambda i: (1, i, 0)),
            pl.BlockSpec((bm, D), lambda i: (i, 0)),
            pl.BlockSpec((D, D), lambda i: (0, 0)),
            pl.BlockSpec((1, D), lambda i: (0, 0)),
            pl.BlockSpec((D, D), lambda i: (0, 0)),
        ],
        out_specs=pl.BlockSpec((bm, D), lambda i: (i, 0)),
        out_shape=jax.ShapeDtypeStruct((N, D), jnp.float32),
    )(aggp, aggp, cntp, cntp, x, wl, bl.reshape(1, D), wr)


def kernel(x, edge_index, Wl1, bl1, Wr1, Wl2, bl2, Wr2):
    e = edge_index.shape[1]
    src = edge_index[0].astype(jnp.int32)
    dst = edge_index[1].astype(jnp.int32)
    nchunks = -(-e // (NW * K))
    nchunks = -(-nchunks // (2 * G)) * (2 * G)
    ng = nchunks // G
    epad = NW * nchunks * K
    srcp = jnp.concatenate([src, jnp.zeros((epad - e,), jnp.int32)])
    dstp = jnp.concatenate([dst, jnp.full((epad - e,), DUMMY, jnp.int32)])
    srcw = srcp.reshape(NW, ng, G, K)
    dstw = dstp.reshape(NW, ng, G, K)

    zrows = jnp.zeros((RPAD, D), jnp.float32)
    ones = jnp.ones((K, D), jnp.float32)

    cntp = _make_cnt(ng)(dstw, zrows, ones)
    aggp1 = _make_agg(ng)(x, srcw, dstw, zrows)
    h = _dense_layer(aggp1, cntp, x, Wl1, bl1, Wr1, relu=True)
    aggp2 = _make_agg(ng)(h, srcw, dstw, zrows)
    out = _dense_layer(aggp2, cntp, h, Wl2, bl2, Wr2, relu=False)
    return out


# speedup vs baseline: 10.2252x; 3.0541x over previous
import functools

import jax
import jax.numpy as jnp
from jax import lax
from jax.experimental import pallas as pl
from jax.experimental.pallas import tpu as pltpu
from jax.experimental.pallas import tpu_sc as plsc

N = 10000
D = 128
NC = 2
NS = 16
NW = NC * NS
K = 128
G = 8
CW = 16
RPAD = 10112
DUMMY = N

_mesh = plsc.VectorSubcoreMesh(core_axis_name="c", subcore_axis_name="s")


def _agg_body(ng, feat, srcw, dstw, zrows, out_acc,
              sb0, db0, sb1, db1, ra, rb, sga, sgb, sst, acc):
    c = lax.axis_index("c")
    s = lax.axis_index("s")
    wid = c * NS + s

    zn = RPAD // NS
    z0 = s * zn
    pltpu.sync_copy(zrows.at[pl.ds(z0, zn)], acc.at[pl.ds(z0, zn)])

    pltpu.sync_copy(srcw.at[wid, 0], sb0)
    pltpu.sync_copy(dstw.at[wid, 0], db0)
    plsc.subcore_barrier()
    pltpu.async_copy(feat.at[sb0.at[0]], ra, sga)

    def run_group(gp, sb, db, sb_nxt, db_nxt, second):
        for j in range(G):
            buf, sem = (ra, sga) if j % 2 == 0 else (rb, sgb)
            obuf, osem = (rb, sgb) if j % 2 == 0 else (ra, sga)
            if j < G - 1:
                pltpu.async_copy(feat.at[sb.at[j + 1]], obuf, osem)
            elif not second:
                pltpu.make_async_copy(srcw.at[wid, 0], sb_nxt, sst).wait()
                pltpu.make_async_copy(dstw.at[wid, 0], db_nxt, sst).wait()
                pltpu.async_copy(feat.at[sb_nxt.at[0]], obuf, osem)
            else:

                @pl.when(gp < ng // 2 - 1)
                def _():
                    pltpu.make_async_copy(srcw.at[wid, 0], sb_nxt, sst).wait()
                    pltpu.make_async_copy(dstw.at[wid, 0], db_nxt, sst).wait()
                    pltpu.async_copy(feat.at[sb_nxt.at[0]], obuf, osem)

            pltpu.make_async_copy(feat.at[sb.at[j]], buf, sem).wait()
            pltpu.sync_copy(buf, acc.at[db.at[j]], add=True)

    def pair(gp, carry):
        g0 = 2 * gp
        pltpu.async_copy(srcw.at[wid, g0 + 1], sb1, sst)
        pltpu.async_copy(dstw.at[wid, g0 + 1], db1, sst)
        run_group(gp, sb0, db0, sb1, db1, second=False)

        @pl.when(gp < ng // 2 - 1)
        def _():
            pltpu.async_copy(srcw.at[wid, g0 + 2], sb0, sst)
            pltpu.async_copy(dstw.at[wid, g0 + 2], db0, sst)

        run_group(gp, sb1, db1, sb0, db0, second=True)
        return carry

    lax.fori_loop(0, ng // 2, pair, 0)
    plsc.subcore_barrier()

    rn = RPAD // NS
    r0 = s * rn
    pltpu.sync_copy(acc.at[pl.ds(r0, rn)], out_acc.at[c, pl.ds(r0, rn)])


def _make_agg(ng):
    scratch = [
        pltpu.VMEM((G, K), jnp.int32),
        pltpu.VMEM((G, K), jnp.int32),
        pltpu.VMEM((G, K), jnp.int32),
        pltpu.VMEM((G, K), jnp.int32),
        pltpu.VMEM((K, D), jnp.float32),
        pltpu.VMEM((K, D), jnp.float32),
        pltpu.SemaphoreType.DMA,
        pltpu.SemaphoreType.DMA,
        pltpu.SemaphoreType.DMA,
        pltpu.VMEM_SHARED((RPAD, D), jnp.float32),
    ]
    return pl.kernel(
        functools.partial(_agg_body, ng),
        out_type=jax.ShapeDtypeStruct((NC, RPAD, D), jnp.float32),
        mesh=_mesh,
        scratch_types=scratch,
    )


def _cnt_body(ng, dstw, zrows, ones_hbm, out_cnt, dfull, ones_v, sd, cnt):
    c = lax.axis_index("c")
    s = lax.axis_index("s")
    wid = c * NS + s

    zn = RPAD // NS
    z0 = s * zn
    pltpu.sync_copy(zrows.at[pl.ds(z0, zn)], cnt.at[pl.ds(z0, zn)])
    pltpu.sync_copy(ones_hbm, ones_v)
    pltpu.sync_copy(dstw.at[wid], dfull)
    plsc.subcore_barrier()

    def group(g, carry):
        for j in range(G):
            pltpu.async_copy(ones_v, cnt.at[dfull.at[g, j]], sd, add=True)
        for j in range(G):
            pltpu.make_async_copy(ones_v, cnt.at[dfull.at[g, j]], sd).wait()
        return carry

    lax.fori_loop(0, ng, group, 0)
    plsc.subcore_barrier()

    rn = RPAD // NS
    r0 = s * rn
    pltpu.sync_copy(cnt.at[pl.ds(r0, rn)], out_cnt.at[c, pl.ds(r0, rn)])


def _make_cnt(ng):
    scratch = [
        pltpu.VMEM((ng, G, K), jnp.int32),
        pltpu.VMEM((K, D), jnp.float32),
        pltpu.SemaphoreType.DMA,
        pltpu.VMEM_SHARED((RPAD, D), jnp.float32),
    ]
    return pl.kernel(
        functools.partial(_cnt_body, ng),
        out_type=jax.ShapeDtypeStruct((NC, RPAD, D), jnp.float32),
        mesh=_mesh,
        scratch_types=scratch,
    )


def _dense_body(relu, a0, a1, c0, c1, x_ref, wl, bl, wr, o_ref):
    agg = a0[0] + a1[0]
    cnt = (c0[0] + c1[0])[:, 0:1]
    mean = agg / jnp.maximum(cnt, 1.0)
    acc = lax.dot_general(mean, wl[...], (((1,), (1,)), ((), ())),
                          precision=lax.Precision.HIGHEST)
    acc += lax.dot_general(x_ref[...], wr[...], (((1,), (1,)), ((), ())),
                           precision=lax.Precision.HIGHEST)
    acc += bl[...]
    o_ref[...] = jnp.maximum(acc, 0.0) if relu else acc


def _dense_layer(aggp, cntp, x, wl, bl, wr, relu):
    bm = 1000
    grid = (N // bm,)
    return pl.pallas_call(
        functools.partial(_dense_body, relu),
        grid=grid,
        in_specs=[
            pl.BlockSpec((1, bm, D), lambda i: (0, i, 0)),
            pl.BlockSpec((1, bm, D), lambda i: (1, i, 0)),
            pl.BlockSpec((1, bm, D), lambda i: (0, i, 0)),
            pl.BlockSpec((1, bm, D), lambda i: (1, i, 0)),
            pl.BlockSpec((bm, D), lambda i: (i, 0)),
            pl.BlockSpec((D, D), lambda i: (0, 0)),
            pl.BlockSpec((1, D), lambda i: (0, 0)),
            pl.BlockSpec((D, D), lambda i: (0, 0)),
        ],
        out_specs=pl.BlockSpec((bm, D), lambda i: (i, 0)),
        out_shape=jax.ShapeDtypeStruct((N, D), jnp.float32),
    )(aggp, aggp, cntp, cntp, x, wl, bl.reshape(1, D), wr)


def kernel(x, edge_index, Wl1, bl1, Wr1, Wl2, bl2, Wr2):
    e = edge_index.shape[1]
    src = edge_index[0].astype(jnp.int32)
    dst = edge_index[1].astype(jnp.int32)
    nchunks = -(-e // (NW * K))
    nchunks = -(-nchunks // (2 * G)) * (2 * G)
    ng = nchunks // G
    epad = NW * nchunks * K
    npd = epad - e
    padsrc = jnp.arange(npd, dtype=jnp.int32) % N
    paddst = DUMMY + jnp.arange(npd, dtype=jnp.int32) % (RPAD - N)
    srcp = jnp.concatenate([src, padsrc])
    dstp = jnp.concatenate([dst, paddst])
    srcw = srcp.reshape(nchunks * K, NW).T.reshape(NW, ng, G, K)
    dstw = dstp.reshape(nchunks * K, NW).T.reshape(NW, ng, G, K)

    zrows = jnp.zeros((RPAD, D), jnp.float32)
    ones = jnp.ones((K, D), jnp.float32)

    cntp = _make_cnt(ng)(dstw, zrows, ones)
    aggp1 = _make_agg(ng)(x, srcw, dstw, zrows)
    h = _dense_layer(aggp1, cntp, x, Wl1, bl1, Wr1, relu=True)
    aggp2 = _make_agg(ng)(h, srcw, dstw, zrows)
    out = _dense_layer(aggp2, cntp, h, Wl2, bl2, Wr2, relu=False)
    return out


# speedup vs baseline: 10.4287x; 1.0199x over previous
import functools

import jax
import jax.numpy as jnp
from jax import lax
from jax.experimental import pallas as pl
from jax.experimental.pallas import tpu as pltpu
from jax.experimental.pallas import tpu_sc as plsc

N = 10000
D = 128
NC = 2
NS = 16
NW = NC * NS
K = 128
G = 8
CW = 16
RPAD = 10112
DUMMY = N

_mesh = plsc.VectorSubcoreMesh(core_axis_name="c", subcore_axis_name="s")


def _agg_body(ng, feat, srcw, dstw, zrows, out_acc,
              sb0, db0, sb1, db1, ra, rb, sga, sgb, sst, acc):
    c = lax.axis_index("c")
    s = lax.axis_index("s")
    wid = c * NS + s

    zn = RPAD // NS
    z0 = s * zn
    pltpu.sync_copy(zrows.at[pl.ds(z0, zn)], acc.at[pl.ds(z0, zn)])

    pltpu.sync_copy(srcw.at[wid, 0], sb0)
    pltpu.sync_copy(dstw.at[wid, 0], db0)
    plsc.subcore_barrier()
    pltpu.async_copy(feat.at[sb0.at[0]], ra, sga)

    def run_group(gp, sb, db, sb_nxt, db_nxt, second):
        for j in range(G):
            buf, sem = (ra, sga) if j % 2 == 0 else (rb, sgb)
            obuf, osem = (rb, sgb) if j % 2 == 0 else (ra, sga)
            if j < G - 1:
                pltpu.async_copy(feat.at[sb.at[j + 1]], obuf, osem)
            elif not second:
                pltpu.make_async_copy(srcw.at[wid, 0], sb_nxt, sst).wait()
                pltpu.make_async_copy(dstw.at[wid, 0], db_nxt, sst).wait()
                pltpu.async_copy(feat.at[sb_nxt.at[0]], obuf, osem)
            else:

                @pl.when(gp < ng // 2 - 1)
                def _():
                    pltpu.make_async_copy(srcw.at[wid, 0], sb_nxt, sst).wait()
                    pltpu.make_async_copy(dstw.at[wid, 0], db_nxt, sst).wait()
                    pltpu.async_copy(feat.at[sb_nxt.at[0]], obuf, osem)

            pltpu.make_async_copy(feat.at[sb.at[j]], buf, sem).wait()
            pltpu.sync_copy(buf, acc.at[db.at[j]], add=True)

    def pair(gp, carry):
        g0 = 2 * gp
        pltpu.async_copy(srcw.at[wid, g0 + 1], sb1, sst)
        pltpu.async_copy(dstw.at[wid, g0 + 1], db1, sst)
        run_group(gp, sb0, db0, sb1, db1, second=False)

        @pl.when(gp < ng // 2 - 1)
        def _():
            pltpu.async_copy(srcw.at[wid, g0 + 2], sb0, sst)
            pltpu.async_copy(dstw.at[wid, g0 + 2], db0, sst)

        run_group(gp, sb1, db1, sb0, db0, second=True)
        return carry

    lax.fori_loop(0, ng // 2, pair, 0)
    plsc.subcore_barrier()

    rn = RPAD // NS
    r0 = s * rn
    pltpu.sync_copy(acc.at[pl.ds(r0, rn)], out_acc.at[c, pl.ds(r0, rn)])


_agg_cache = {}


def _make_agg(ng):
    if ng in _agg_cache:
        return _agg_cache[ng]
    scratch = [
        pltpu.VMEM((G, K), jnp.int32),
        pltpu.VMEM((G, K), jnp.int32),
        pltpu.VMEM((G, K), jnp.int32),
        pltpu.VMEM((G, K), jnp.int32),
        pltpu.VMEM((K, D), jnp.float32),
        pltpu.VMEM((K, D), jnp.float32),
        pltpu.SemaphoreType.DMA,
        pltpu.SemaphoreType.DMA,
        pltpu.SemaphoreType.DMA,
        pltpu.VMEM_SHARED((RPAD, D), jnp.float32),
    ]
    _agg_cache[ng] = pl.kernel(
        functools.partial(_agg_body, ng),
        out_type=jax.ShapeDtypeStruct((NC, RPAD, D), jnp.float32),
        mesh=_mesh,
        scratch_types=scratch,
    )
    return _agg_cache[ng]


def _cnt_body(ng, dstw, zrows, ones_hbm, out_cnt, dfull, ones_v, sd, cnt):
    c = lax.axis_index("c")
    s = lax.axis_index("s")
    wid = c * NS + s

    zn = RPAD // NS
    z0 = s * zn
    pltpu.sync_copy(zrows.at[pl.ds(z0, zn)], cnt.at[pl.ds(z0, zn)])
    pltpu.sync_copy(ones_hbm, ones_v)
    pltpu.sync_copy(dstw.at[wid], dfull)
    plsc.subcore_barrier()

    def group(g, carry):
        for j in range(G):
            pltpu.async_copy(ones_v, cnt.at[dfull.at[g, j]], sd, add=True)
        for j in range(G):
            pltpu.make_async_copy(ones_v, cnt.at[dfull.at[g, j]], sd).wait()
        return carry

    lax.fori_loop(0, ng, group, 0)
    plsc.subcore_barrier()

    rn = RPAD // NS
    r0 = s * rn
    pltpu.sync_copy(cnt.at[pl.ds(r0, rn)], out_cnt.at[c, pl.ds(r0, rn)])


def _make_cnt(ng):
    scratch = [
        pltpu.VMEM((ng, G, K), jnp.int32),
        pltpu.VMEM((K, D), jnp.float32),
        pltpu.SemaphoreType.DMA,
        pltpu.VMEM_SHARED((RPAD, D), jnp.float32),
    ]
    return pl.kernel(
        functools.partial(_cnt_body, ng),
        out_type=jax.ShapeDtypeStruct((NC, RPAD, D), jnp.float32),
        mesh=_mesh,
        scratch_types=scratch,
    )


def _dense_body(relu, a0, a1, c0, c1, x_ref, wl, bl, wr, o_ref):
    agg = a0[0] + a1[0]
    cnt = (c0[0] + c1[0])[:, 0:1]
    mean = agg / jnp.maximum(cnt, 1.0)
    acc = lax.dot_general(mean, wl[...], (((1,), (1,)), ((), ())),
                          precision=lax.Precision.HIGHEST)
    acc += lax.dot_general(x_ref[...], wr[...], (((1,), (1,)), ((), ())),
                           precision=lax.Precision.HIGHEST)
    acc += bl[...]
    o_ref[...] = jnp.maximum(acc, 0.0) if relu else acc


def _dense_layer(aggp, cntp, x, wl, bl, wr, relu):
    bm = 1000
    grid = (N // bm,)
    return pl.pallas_call(
        functools.partial(_dense_body, relu),
        grid=grid,
        in_specs=[
            pl.BlockSpec((1, bm, D), lambda i: (0, i, 0)),
            pl.BlockSpec((1, bm, D), lambda i: (1, i, 0)),
            pl.BlockSpec((1, bm, D), lambda i: (0, i, 0)),
            pl.BlockSpec((1, bm, D), lambda i: (1, i, 0)),
            pl.BlockSpec((bm, D), lambda i: (i, 0)),
            pl.BlockSpec((D, D), lambda i: (0, 0)),
            pl.BlockSpec((1, D), lambda i: (0, 0)),
            pl.BlockSpec((D, D), lambda i: (0, 0)),
        ],
        out_specs=pl.BlockSpec((bm, D), lambda i: (i, 0)),
        out_shape=jax.ShapeDtypeStruct((N, D), jnp.float32),
    )(aggp, aggp, cntp, cntp, x, wl, bl.reshape(1, D), wr)


def kernel(x, edge_index, Wl1, bl1, Wr1, Wl2, bl2, Wr2):
    e = edge_index.shape[1]
    src = edge_index[0].astype(jnp.int32)
    dst = edge_index[1].astype(jnp.int32)
    nchunks = -(-e // (NW * K))
    nchunks = -(-nchunks // (2 * G)) * (2 * G)
    ng = nchunks // G
    epad = NW * nchunks * K
    npd = epad - e
    padsrc = jnp.arange(npd, dtype=jnp.int32) % N
    paddst = DUMMY + jnp.arange(npd, dtype=jnp.int32) % (RPAD - N)
    srcp = jnp.concatenate([src, padsrc])
    dstp = jnp.concatenate([dst, paddst])
    srcw = srcp.reshape(NW, ng, G, K)
    dstw = dstp.reshape(NW, ng, G, K)

    zrows = jnp.zeros((RPAD, D), jnp.float32)
    ones = jnp.ones((K, D), jnp.float32)

    cntp = _make_cnt(ng)(dstw, zrows, ones)
    aggp1 = _make_agg(ng)(x, srcw, dstw, zrows)
    h = _dense_layer(aggp1, cntp, x, Wl1, bl1, Wr1, relu=True)
    aggp2 = _make_agg(ng)(h, srcw, dstw, zrows)
    out = _dense_layer(aggp2, cntp, h, Wl2, bl2, Wr2, relu=False)
    return out


# speedup vs baseline: 13.2497x; 1.2705x over previous
import functools

import jax
import jax.numpy as jnp
from jax import lax
from jax.experimental import pallas as pl
from jax.experimental.pallas import tpu as pltpu
from jax.experimental.pallas import tpu_sc as plsc

N = 10000
D = 128
NC = 2
NS = 16
NW = NC * NS
K = 128
G = 8
RPAD = 10112
DUMMY = N

_mesh = plsc.VectorSubcoreMesh(core_axis_name="c", subcore_axis_name="s")


def _agg_body(ng, want_cnt, *refs):
    if want_cnt:
        (feat, srcw, dstw, zrows, zcnt1, ones_hbm, out_acc, out_c0, out_c1,
         sb0, db0, sb1, db1, ra, rb, ones_v, bounce,
         sga, sgb, sst, sd, acc, cnt1) = refs
    else:
        (feat, srcw, dstw, zrows, out_acc,
         sb0, db0, sb1, db1, ra, rb, sga, sgb, sst, acc) = refs
    c = lax.axis_index("c")
    s = lax.axis_index("s")
    wid = c * NS + s

    zn = RPAD // NS
    z0 = s * zn
    pltpu.sync_copy(zrows, ra)
    for piece in range(4):
        pltpu.sync_copy(ra, acc.at[pl.ds(z0 + piece * K, K)])
    pltpu.sync_copy(ra.at[pl.ds(0, zn - 4 * K)],
                    acc.at[pl.ds(z0 + 4 * K, zn - 4 * K)])
    if want_cnt:
        pltpu.sync_copy(zcnt1, bounce)
        pltpu.sync_copy(bounce, cnt1.at[pl.ds(z0, zn)])
        pltpu.sync_copy(ones_hbm, ones_v)

    pltpu.sync_copy(srcw.at[wid, 0], sb0)
    pltpu.sync_copy(dstw.at[wid, 0], db0)
    plsc.subcore_barrier()
    pltpu.async_copy(feat.at[sb0.at[0]], ra, sga)

    def run_group(gp, sb, db, sb_nxt, db_nxt, second):
        for j in range(G):
            buf, sem = (ra, sga) if j % 2 == 0 else (rb, sgb)
            obuf, osem = (rb, sgb) if j % 2 == 0 else (ra, sga)
            if j < G - 1:
                pltpu.async_copy(feat.at[sb.at[j + 1]], obuf, osem)
            elif not second:
                pltpu.make_async_copy(srcw.at[wid, 0], sb_nxt, sst).wait()
                pltpu.make_async_copy(dstw.at[wid, 0], db_nxt, sst).wait()
                pltpu.async_copy(feat.at[sb_nxt.at[0]], obuf, osem)
            else:

                @pl.when(gp < ng // 2 - 1)
                def _():
                    pltpu.make_async_copy(srcw.at[wid, 0], sb_nxt, sst).wait()
                    pltpu.make_async_copy(dstw.at[wid, 0], db_nxt, sst).wait()
                    pltpu.async_copy(feat.at[sb_nxt.at[0]], obuf, osem)

            if want_cnt:
                pltpu.async_copy(ones_v, cnt1.at[db.at[j]], sd, add=True)
            pltpu.make_async_copy(feat.at[sb.at[j]], buf, sem).wait()
            pltpu.sync_copy(buf, acc.at[db.at[j]], add=True)
        if want_cnt:
            for j in range(G):
                pltpu.make_async_copy(ones_v, cnt1.at[db.at[j]], sd).wait()

    def pair(gp, carry):
        g0 = 2 * gp
        pltpu.async_copy(srcw.at[wid, g0 + 1], sb1, sst)
        pltpu.async_copy(dstw.at[wid, g0 + 1], db1, sst)
        run_group(gp, sb0, db0, sb1, db1, second=False)

        @pl.when(gp < ng // 2 - 1)
        def _():
            pltpu.async_copy(srcw.at[wid, g0 + 2], sb0, sst)
            pltpu.async_copy(dstw.at[wid, g0 + 2], db0, sst)

        run_group(gp, sb1, db1, sb0, db0, second=True)
        return carry

    lax.fori_loop(0, ng // 2, pair, 0)
    plsc.subcore_barrier()

    rn = RPAD // NS
    r0 = s * rn
    pltpu.sync_copy(acc.at[pl.ds(r0, rn)], out_acc.at[c, pl.ds(r0, rn)])
    if want_cnt:
        pltpu.sync_copy(cnt1.at[pl.ds(r0, rn)], bounce)

        @pl.when(c == 0)
        def _():
            pltpu.sync_copy(bounce, out_c0.at[pl.ds(r0, rn)])

        @pl.when(c == 1)
        def _():
            pltpu.sync_copy(bounce, out_c1.at[pl.ds(r0, rn)])


_agg_cache = {}


def _make_agg(ng, want_cnt):
    key = (ng, want_cnt)
    if key in _agg_cache:
        return _agg_cache[key]
    outs = jax.ShapeDtypeStruct((NC, RPAD, D), jnp.float32)
    scratch = [
        pltpu.VMEM((G, K), jnp.int32),
        pltpu.VMEM((G, K), jnp.int32),
        pltpu.VMEM((G, K), jnp.int32),
        pltpu.VMEM((G, K), jnp.int32),
        pltpu.VMEM((K, D), jnp.float32),
        pltpu.VMEM((K, D), jnp.float32),
    ]
    if want_cnt:
        outs = (outs,
                jax.ShapeDtypeStruct((RPAD,), jnp.float32),
                jax.ShapeDtypeStruct((RPAD,), jnp.float32))
        scratch += [
            pltpu.VMEM((K,), jnp.float32),
            pltpu.VMEM((RPAD // NS,), jnp.float32),
        ]
    scratch += [
        pltpu.SemaphoreType.DMA,
        pltpu.SemaphoreType.DMA,
        pltpu.SemaphoreType.DMA,
    ]
    if want_cnt:
        scratch.append(pltpu.SemaphoreType.DMA)
    scratch.append(pltpu.VMEM_SHARED((RPAD, D), jnp.float32))
    if want_cnt:
        scratch.append(pltpu.VMEM_SHARED((RPAD,), jnp.float32))
    _agg_cache[key] = pl.kernel(
        functools.partial(_agg_body, ng, want_cnt),
        out_type=outs,
        mesh=_mesh,
        scratch_types=scratch,
    )
    return _agg_cache[key]


def _dense_body(relu, a0, a1, c_ref, x_ref, wl, bl, wr, o_ref):
    agg = a0[0] + a1[0]
    cnt = c_ref[...]
    mean = agg / jnp.maximum(cnt, 1.0)
    acc = lax.dot_general(mean, wl[...], (((1,), (1,)), ((), ())),
                          precision=lax.Precision.HIGHEST)
    acc += lax.dot_general(x_ref[...], wr[...], (((1,), (1,)), ((), ())),
                           precision=lax.Precision.HIGHEST)
    acc += bl[...]
    o_ref[...] = jnp.maximum(acc, 0.0) if relu else acc


def _dense_layer(aggp, cntp, x, wl, bl, wr, relu):
    bm = 2000
    grid = (N // bm,)
    return pl.pallas_call(
        functools.partial(_dense_body, relu),
        grid=grid,
        in_specs=[
            pl.BlockSpec((1, bm, D), lambda i: (0, i, 0)),
            pl.BlockSpec((1, bm, D), lambda i: (1, i, 0)),
            pl.BlockSpec((bm, 1), lambda i: (i, 0)),
            pl.BlockSpec((bm, D), lambda i: (i, 0)),
            pl.BlockSpec((D, D), lambda i: (0, 0)),
            pl.BlockSpec((1, D), lambda i: (0, 0)),
            pl.BlockSpec((D, D), lambda i: (0, 0)),
        ],
        out_specs=pl.BlockSpec((bm, D), lambda i: (i, 0)),
        out_shape=jax.ShapeDtypeStruct((N, D), jnp.float32),
    )(aggp, aggp, cntp, x, wl, bl.reshape(1, D), wr)


def kernel(x, edge_index, Wl1, bl1, Wr1, Wl2, bl2, Wr2):
    e = edge_index.shape[1]
    src = edge_index[0].astype(jnp.int32)
    dst = edge_index[1].astype(jnp.int32)
    nchunks = -(-e // (NW * K))
    nchunks = -(-nchunks // (2 * G)) * (2 * G)
    ng = nchunks // G
    epad = NW * nchunks * K
    npd = epad - e
    padsrc = jnp.arange(npd, dtype=jnp.int32) % N
    paddst = DUMMY + jnp.arange(npd, dtype=jnp.int32) % (RPAD - N)
    srcw = jnp.concatenate([src, padsrc]).reshape(NW, ng, G, K)
    dstw = jnp.concatenate([dst, paddst]).reshape(NW, ng, G, K)

    zrows = jnp.zeros((K, D), jnp.float32)
    zcnt1 = jnp.zeros((RPAD // NS,), jnp.float32)
    ones = jnp.ones((K,), jnp.float32)

    aggp1, cnt0, cnt1 = _make_agg(ng, True)(x, srcw, dstw, zrows, zcnt1, ones)
    cntp = (cnt0 + cnt1).reshape(RPAD, 1)
    h = _dense_layer(aggp1, cntp, x, Wl1, bl1, Wr1, relu=True)
    aggp2 = _make_agg(ng, False)(h, srcw, dstw, zrows)
    out = _dense_layer(aggp2, cntp, h, Wl2, bl2, Wr2, relu=False)
    return out


# speedup vs baseline: 13.2890x; 1.0030x over previous
import functools

import jax
import jax.numpy as jnp
from jax import lax
from jax.experimental import pallas as pl
from jax.experimental.pallas import tpu as pltpu
from jax.experimental.pallas import tpu_sc as plsc

N = 10000
D = 128
NC = 2
NS = 16
NW = NC * NS
K = 128
G = 8
RPAD = 10112
DUMMY = N

_mesh = plsc.VectorSubcoreMesh(core_axis_name="c", subcore_axis_name="s")


def _agg_body(ng, want_cnt, *refs):
    if want_cnt:
        (feat, srcw, dstw, zrows, zcnt1, ones_hbm, out_acc, out_c0, out_c1,
         sb0, db0, sb1, db1, ra, rb, ones_v, bounce,
         sga, sgb, sst, sd, acc, cnt1) = refs
    else:
        (feat, srcw, dstw, zrows, out_acc,
         sb0, db0, sb1, db1, ra, rb, sga, sgb, sst, acc) = refs
    c = lax.axis_index("c")
    s = lax.axis_index("s")
    wid = c * NS + s

    zn = RPAD // NS
    z0 = s * zn
    pltpu.sync_copy(zrows, ra)
    for piece in range(4):
        pltpu.sync_copy(ra, acc.at[pl.ds(z0 + piece * K, K)])
    pltpu.sync_copy(ra.at[pl.ds(0, zn - 4 * K)],
                    acc.at[pl.ds(z0 + 4 * K, zn - 4 * K)])
    if want_cnt:
        pltpu.sync_copy(zcnt1, bounce)
        pltpu.sync_copy(bounce, cnt1.at[pl.ds(z0, zn)])
        pltpu.sync_copy(ones_hbm, ones_v)

    pltpu.sync_copy(srcw.at[wid, 0], sb0)
    pltpu.sync_copy(dstw.at[wid, 0], db0)
    plsc.subcore_barrier()
    pltpu.async_copy(feat.at[sb0.at[0]], ra, sga)

    def run_group(gp, sb, db, sb_nxt, db_nxt, second):
        for j in range(G):
            buf, sem = (ra, sga) if j % 2 == 0 else (rb, sgb)
            obuf, osem = (rb, sgb) if j % 2 == 0 else (ra, sga)
            if j < G - 1:
                pltpu.async_copy(feat.at[sb.at[j + 1]], obuf, osem)
            elif not second:
                pltpu.make_async_copy(srcw.at[wid, 0], sb_nxt, sst).wait()
                pltpu.make_async_copy(dstw.at[wid, 0], db_nxt, sst).wait()
                pltpu.async_copy(feat.at[sb_nxt.at[0]], obuf, osem)
            else:

                @pl.when(gp < ng // 2 - 1)
                def _():
                    pltpu.make_async_copy(srcw.at[wid, 0], sb_nxt, sst).wait()
                    pltpu.make_async_copy(dstw.at[wid, 0], db_nxt, sst).wait()
                    pltpu.async_copy(feat.at[sb_nxt.at[0]], obuf, osem)

            if want_cnt:
                pltpu.async_copy(ones_v, cnt1.at[db.at[j]], sd, add=True)
            pltpu.make_async_copy(feat.at[sb.at[j]], buf, sem).wait()
            pltpu.sync_copy(buf, acc.at[db.at[j]], add=True)
        if want_cnt:
            for j in range(G):
                pltpu.make_async_copy(ones_v, cnt1.at[db.at[j]], sd).wait()

    def pair(gp, carry):
        g0 = 2 * gp
        pltpu.async_copy(srcw.at[wid, g0 + 1], sb1, sst)
        pltpu.async_copy(dstw.at[wid, g0 + 1], db1, sst)
        run_group(gp, sb0, db0, sb1, db1, second=False)

        @pl.when(gp < ng // 2 - 1)
        def _():
            pltpu.async_copy(srcw.at[wid, g0 + 2], sb0, sst)
            pltpu.async_copy(dstw.at[wid, g0 + 2], db0, sst)

        run_group(gp, sb1, db1, sb0, db0, second=True)
        return carry

    lax.fori_loop(0, ng // 2, pair, 0)
    plsc.subcore_barrier()

    rn = RPAD // NS
    r0 = s * rn
    pltpu.sync_copy(acc.at[pl.ds(r0, rn)], out_acc.at[c, pl.ds(r0, rn)])
    if want_cnt:
        pltpu.sync_copy(cnt1.at[pl.ds(r0, rn)], bounce)

        @pl.when(c == 0)
        def _():
            pltpu.sync_copy(bounce, out_c0.at[pl.ds(r0, rn)])

        @pl.when(c == 1)
        def _():
            pltpu.sync_copy(bounce, out_c1.at[pl.ds(r0, rn)])


_agg_cache = {}


def _make_agg(ng, want_cnt):
    key = (ng, want_cnt)
    if key in _agg_cache:
        return _agg_cache[key]
    outs = jax.ShapeDtypeStruct((NC, RPAD, D), jnp.float32)
    scratch = [
        pltpu.VMEM((G, K), jnp.int32),
        pltpu.VMEM((G, K), jnp.int32),
        pltpu.VMEM((G, K), jnp.int32),
        pltpu.VMEM((G, K), jnp.int32),
        pltpu.VMEM((K, D), jnp.float32),
        pltpu.VMEM((K, D), jnp.float32),
    ]
    if want_cnt:
        outs = (outs,
                jax.ShapeDtypeStruct((RPAD,), jnp.float32),
                jax.ShapeDtypeStruct((RPAD,), jnp.float32))
        scratch += [
            pltpu.VMEM((K,), jnp.float32),
            pltpu.VMEM((RPAD // NS,), jnp.float32),
        ]
    scratch += [
        pltpu.SemaphoreType.DMA,
        pltpu.SemaphoreType.DMA,
        pltpu.SemaphoreType.DMA,
    ]
    if want_cnt:
        scratch.append(pltpu.SemaphoreType.DMA)
    scratch.append(pltpu.VMEM_SHARED((RPAD, D), jnp.float32))
    if want_cnt:
        scratch.append(pltpu.VMEM_SHARED((RPAD,), jnp.float32))
    _agg_cache[key] = pl.kernel(
        functools.partial(_agg_body, ng, want_cnt),
        out_type=outs,
        mesh=_mesh,
        scratch_types=scratch,
    )
    return _agg_cache[key]


def _dense_body(relu, a0, a1, c_ref, x_ref, wl, bl, wr, o_ref):
    agg = a0[0] + a1[0]
    cnt = c_ref[...]
    mean = agg / jnp.maximum(cnt, 1.0)
    acc = lax.dot_general(mean, wl[...], (((1,), (1,)), ((), ())),
                          precision=lax.Precision.HIGHEST)
    acc += lax.dot_general(x_ref[...], wr[...], (((1,), (1,)), ((), ())),
                           precision=lax.Precision.HIGHEST)
    acc += bl[...]
    o_ref[...] = jnp.maximum(acc, 0.0) if relu else acc


def _dense_layer(aggp, cntp, x, wl, bl, wr, relu):
    bm = 2000
    grid = (N // bm,)
    return pl.pallas_call(
        functools.partial(_dense_body, relu),
        grid=grid,
        in_specs=[
            pl.BlockSpec((1, bm, D), lambda i: (0, i, 0)),
            pl.BlockSpec((1, bm, D), lambda i: (1, i, 0)),
            pl.BlockSpec((bm, 1), lambda i: (i, 0)),
            pl.BlockSpec((bm, D), lambda i: (i, 0)),
            pl.BlockSpec((D, D), lambda i: (0, 0)),
            pl.BlockSpec((1, D), lambda i: (0, 0)),
            pl.BlockSpec((D, D), lambda i: (0, 0)),
        ],
        out_specs=pl.BlockSpec((bm, D), lambda i: (i, 0)),
        out_shape=jax.ShapeDtypeStruct((N, D), jnp.float32),
    )(aggp, aggp, cntp, x, wl, bl.reshape(1, D), wr)


def kernel(x, edge_index, Wl1, bl1, Wr1, Wl2, bl2, Wr2):
    e = edge_index.shape[1]
    ei = edge_index.astype(jnp.int32)
    nchunks = -(-e // (NW * K))
    nchunks = -(-nchunks // (2 * G)) * (2 * G)
    ng = nchunks // G
    epad = NW * nchunks * K
    npd = epad - e
    padsrc = jnp.arange(npd, dtype=jnp.int32) % N
    paddst = DUMMY + jnp.arange(npd, dtype=jnp.int32) % (RPAD - N)
    eiw = jnp.concatenate([ei, jnp.stack([padsrc, paddst])],
                          axis=1).reshape(2, NW, ng, G, K)
    srcw = eiw[0]
    dstw = eiw[1]

    zrows = jnp.zeros((K, D), jnp.float32)
    zcnt1 = jnp.zeros((RPAD // NS,), jnp.float32)
    ones = jnp.ones((K,), jnp.float32)

    aggp1, cnt0, cnt1 = _make_agg(ng, True)(x, srcw, dstw, zrows, zcnt1, ones)
    cntp = (cnt0 + cnt1).reshape(RPAD, 1)
    h = _dense_layer(aggp1, cntp, x, Wl1, bl1, Wr1, relu=True)
    aggp2 = _make_agg(ng, False)(h, srcw, dstw, zrows)
    out = _dense_layer(aggp2, cntp, h, Wl2, bl2, Wr2, relu=False)
    return out
